# Initial kernel scaffold; baseline (speedup 1.0000x reference)
import jax
import jax.numpy as jnp
from jax.experimental import pallas as pl


def kernel(x, w1, w2, router_w, router_b):
    raise NotImplementedError("write your pallas kernel here")


# speedup vs baseline: 2.0835x; 2.0835x over previous
import functools

import jax
import jax.numpy as jnp
from jax import lax
from jax.experimental import pallas as pl
from jax.experimental.pallas import tpu as pltpu
from jax.experimental.pallas import tpu_sc as plsc

D_MODEL = 768
D_FF = 3072
N_EXP = 8
T_TOK = 2048
TILE_M = 128
N_TILES = (T_TOK + N_EXP * TILE_M) // TILE_M
M_PAD = N_TILES * TILE_M
SCALE = 3e-06


def _router_body(x_ref, rw_ref, rb_ref,
                 xs_ref, dest_ref, te_ref, act_ref, loss_ref):
    x = x_ref[...]
    logits = jnp.dot(x, rw_ref[...], preferred_element_type=jnp.float32)
    logits = logits + rb_ref[...]
    lmax = jnp.max(logits, axis=1, keepdims=True)
    p = jnp.exp(logits - lmax)
    probs = p / jnp.sum(p, axis=1, keepdims=True)
    maxp = jnp.max(probs, axis=1, keepdims=True)
    cols = lax.broadcasted_iota(jnp.int32, (T_TOK, N_EXP), 1).astype(jnp.float32)
    eid = jnp.min(jnp.where(probs == maxp, cols, jnp.float32(N_EXP)),
                  axis=1, keepdims=True)
    onehot = (cols == eid).astype(jnp.float32)
    counts = jnp.sum(onehot, axis=0, keepdims=True)

    CH = 256
    r_i = lax.broadcasted_iota(jnp.int32, (CH, CH), 0)
    c_i = lax.broadcasted_iota(jnp.int32, (CH, CH), 1)
    tril = (r_i > c_i).astype(jnp.float32)
    carry = jnp.zeros((1, N_EXP), jnp.float32)
    rank_chunks = []
    for c in range(T_TOK // CH):
        oh = onehot[c * CH:(c + 1) * CH]
        r = jnp.dot(tril, oh, preferred_element_type=jnp.float32) + carry
        rank_chunks.append(jnp.sum(r * oh, axis=1, keepdims=True))
        carry = carry + jnp.sum(oh, axis=0, keepdims=True)
    rank = jnp.concatenate(rank_chunks, axis=0)

    pc = jnp.ceil(counts / TILE_M) * TILE_M
    a_i = lax.broadcasted_iota(jnp.int32, (N_EXP, N_EXP), 0)
    b_i = lax.broadcasted_iota(jnp.int32, (N_EXP, N_EXP), 1)
    excl = (a_i < b_i).astype(jnp.float32)
    offs = jnp.dot(pc, excl, preferred_element_type=jnp.float32)
    dest = jnp.sum(onehot * offs, axis=1, keepdims=True) + rank
    dest_ref[...] = dest.astype(jnp.int32)

    xs_ref[...] = x * maxp

    n_active = jnp.sum(pc) / TILE_M
    k_i = lax.broadcasted_iota(jnp.int32, (N_TILES, 1), 0).astype(jnp.float32)
    kk = jnp.minimum(k_i, n_active - 1.0)
    te = jnp.sum((kk * TILE_M >= offs).astype(jnp.float32), axis=1,
                 keepdims=True) - 1.0
    te_ref[...] = te.astype(jnp.int32)
    act_ref[...] = (k_i < n_active).astype(jnp.int32)

    psum = jnp.sum(onehot * maxp, axis=0, keepdims=True)
    loss = jnp.sum((counts / T_TOK) * (psum / (T_TOK * T_TOK)),
                   axis=1, keepdims=True)
    loss_ref[...] = loss * (SCALE * N_EXP)


def _ffn_body(te_ref, act_ref, xs_ref, w1_ref, w2_ref, out_ref):
    k = pl.program_id(0)

    @pl.when(act_ref[k] == 1)
    def _():
        h = jnp.dot(xs_ref[...], w1_ref[0],
                    preferred_element_type=jnp.float32)
        h = jnp.maximum(h, 0.0)
        out_ref[...] = jnp.dot(h, w2_ref[0],
                               preferred_element_type=jnp.float32)


def _sc_permute(gather: bool, n_rows_out: int):
    info = plsc.get_sparse_core_info()
    nc, ns = info.num_cores, info.num_subcores
    nw = nc * ns
    bw = T_TOK // nw
    mesh = plsc.VectorSubcoreMesh(core_axis_name="c", subcore_axis_name="s")

    @functools.partial(
        pl.kernel, mesh=mesh,
        out_type=jax.ShapeDtypeStruct((n_rows_out, D_MODEL), jnp.float32),
        scratch_types=[
            pltpu.VMEM((bw,), jnp.int32),
            pltpu.VMEM((bw, D_MODEL), jnp.float32),
            pltpu.SemaphoreType.DMA,
        ],
    )
    def body(rows_hbm, idx_hbm, out_hbm, idx_v, rows_v, sem):
        wid = lax.axis_index("s") * nc + lax.axis_index("c")
        base = wid * bw
        pltpu.sync_copy(idx_hbm.at[pl.ds(base, bw)], idx_v)
        if gather:
            pltpu.async_copy(rows_hbm.at[idx_v], rows_v, sem).wait()
            pltpu.sync_copy(rows_v, out_hbm.at[pl.ds(base, bw)])
        else:
            pltpu.sync_copy(rows_hbm.at[pl.ds(base, bw)], rows_v)
            pltpu.async_copy(rows_v, out_hbm.at[idx_v], sem).wait()

    return body


def kernel(x, w1, w2, router_w, router_b):
    xf = x.reshape(T_TOK, D_MODEL)

    xs, dest, te, act, loss = pl.pallas_call(
        _router_body,
        out_shape=[
            jax.ShapeDtypeStruct((T_TOK, D_MODEL), jnp.float32),
            jax.ShapeDtypeStruct((T_TOK, 1), jnp.int32),
            jax.ShapeDtypeStruct((N_TILES, 1), jnp.int32),
            jax.ShapeDtypeStruct((N_TILES, 1), jnp.int32),
            jax.ShapeDtypeStruct((1, 1), jnp.float32),
        ],
    )(xf, router_w, router_b.reshape(1, N_EXP))

    dest = dest.reshape(T_TOK)
    x_sorted = _sc_permute(gather=False, n_rows_out=M_PAD)(xs, dest)

    grid_spec = pltpu.PrefetchScalarGridSpec(
        num_scalar_prefetch=2,
        grid=(N_TILES,),
        in_specs=[
            pl.BlockSpec((TILE_M, D_MODEL), lambda k, te, act: (k, 0)),
            pl.BlockSpec((1, D_MODEL, D_FF), lambda k, te, act: (te[k], 0, 0)),
            pl.BlockSpec((1, D_FF, D_MODEL), lambda k, te, act: (te[k], 0, 0)),
        ],
        out_specs=pl.BlockSpec((TILE_M, D_MODEL), lambda k, te, act: (k, 0)),
    )
    out_sorted = pl.pallas_call(
        _ffn_body,
        grid_spec=grid_spec,
        out_shape=jax.ShapeDtypeStruct((M_PAD, D_MODEL), jnp.float32),
    )(te.reshape(N_TILES), act.reshape(N_TILES), x_sorted, w1, w2)

    out = _sc_permute(gather=True, n_rows_out=T_TOK)(out_sorted, dest)
    return out.reshape(1, T_TOK, D_MODEL), loss.reshape(())
